# baseline (speedup 1.0000x reference)
import functools

import jax
import jax.numpy as jnp
from jax import lax
from jax.experimental import pallas as pl
from jax.experimental.pallas import tpu as pltpu
from jax.experimental.pallas import tpu_sc as plsc

N = 10000
NP = 10240
E = 320000
D = 128
DO = 64

NC = 2
NS = 16
L = 16
NW = NC * NS
EPW = E // NW
NPT = 640
NPT_LAST = N - (NS - 1) * NPT
CH = 80
NCH = EPW // CH

_MESH = plsc.VectorSubcoreMesh(core_axis_name="c", subcore_axis_name="s")
_SC_PARAMS = pltpu.CompilerParams(needs_layout_passes=False)


def _deg_body(ei_hbm, parts_hbm, dst_v, deg_v):
    c = lax.axis_index("c")
    s = lax.axis_index("s")
    wid = s * NC + c

    zl = jnp.zeros((L,), jnp.float32)

    def zero(i, _):
        for k in range(4):
            deg_v[pl.ds((i * 4 + k) * L, L)] = zl
        return 0

    lax.fori_loop(0, NP // L // 4, zero, 0)

    pltpu.sync_copy(ei_hbm.at[pl.ds(E + wid * EPW, EPW)], dst_v)
    ones = jnp.ones((L,), jnp.float32)

    def scat(i, _):
        for k in range(5):
            idx = dst_v[pl.ds((i * 5 + k) * L, L)]
            plsc.addupdate_scatter(deg_v, [idx], ones)
        return 0

    lax.fori_loop(0, EPW // L // 5, scat, 0)
    pltpu.sync_copy(deg_v, parts_hbm.at[wid])


_deg_kernel = functools.partial(
    pl.kernel,
    out_type=jax.ShapeDtypeStruct((NW, NP), jnp.float32),
    mesh=_MESH,
    compiler_params=_SC_PARAMS,
    scratch_types=[
        pltpu.VMEM((EPW,), jnp.int32),
        pltpu.VMEM((NP,), jnp.float32),
    ],
)(_deg_body)


def _spmm_body(hs_hbm, ei_hbm, out_hbm,
               src0_v, src1_v, src2_v, dst0_v, dst1_v, dst2_v,
               rows0_v, rows1_v, rows2_v,
               gsem0, gsem1, gsem2, isem0, isem1, isem2, acc):
    c = lax.axis_index("c")
    s = lax.axis_index("s")
    wid = s * NC + c
    row0 = pl.multiple_of(s * NPT, 8)

    @pl.when(jnp.logical_and(c == 0, s < NS - 1))
    def _():
        pltpu.sync_copy(hs_hbm.at[pl.ds(row0, NPT)], acc.at[pl.ds(row0, NPT)])

    @pl.when(jnp.logical_and(c == 0, s == NS - 1))
    def _():
        pltpu.sync_copy(hs_hbm.at[pl.ds(row0, NPT_LAST)],
                        acc.at[pl.ds(row0, NPT_LAST)])

    @pl.when(c != 0)
    def _():
        zl = jnp.zeros((L,), jnp.float32)

        def zrow(i, _):
            r = i // (D // L)
            k = i % (D // L)
            rows0_v[r, pl.ds(k * L, L)] = zl
            return 0

        lax.fori_loop(0, CH * D // L, zrow, 0)

    @pl.when(jnp.logical_and(c != 0, s < NS - 1))
    def _():
        for t in range(NPT // CH):
            r = pl.multiple_of(row0 + t * CH, 8)
            pltpu.sync_copy(rows0_v, acc.at[pl.ds(r, CH)])

    @pl.when(jnp.logical_and(c != 0, s == NS - 1))
    def _():
        for t in range(NPT_LAST // CH):
            r = pl.multiple_of(row0 + t * CH, 8)
            pltpu.sync_copy(rows0_v, acc.at[pl.ds(r, CH)])

    plsc.subcore_barrier()

    bufs = (rows0_v, rows1_v, rows2_v)
    gsems = (gsem0, gsem1, gsem2)
    sbufs = (src0_v, src1_v, src2_v)
    dbufs = (dst0_v, dst1_v, dst2_v)
    isems = (isem0, isem1, isem2)

    def srccopy(i, b):
        off = pl.multiple_of(wid * EPW + i * CH, 8)
        return pltpu.make_async_copy(ei_hbm.at[pl.ds(off, CH)],
                                     sbufs[b], isems[b])

    def dstcopy(i, b):
        off = pl.multiple_of(E + wid * EPW + i * CH, 8)
        return pltpu.make_async_copy(ei_hbm.at[pl.ds(off, CH)],
                                     dbufs[b], isems[b])

    def idxload(i, b):
        srccopy(i, b).start()
        dstcopy(i, b).start()

    def idx_wait(i, b):
        srccopy(i, b).wait()
        dstcopy(i, b).wait()

    def gather(i, b):
        return pltpu.make_async_copy(hs_hbm.at[sbufs[b]], bufs[b], gsems[b])

    idxload(0, 0)
    idxload(1, 1)
    idx_wait(0, 0)
    gather(0, 0).start()
    idx_wait(1, 1)
    gather(1, 1).start()
    idxload(2, 2)

    def tri(j, _):
        for k in range(3):
            i = j * 3 + k
            b, b2 = k, (k + 2) % 3

            @pl.when(i + 2 < NCH)
            def _():
                idx_wait(i + 2, b2)
                gather(i + 2, b2).start()

            @pl.when(i < NCH)
            def _():
                gather(i, b).wait()
                pltpu.sync_copy(bufs[b], acc.at[dbufs[b]], add=True)

            @pl.when(i + 3 < NCH)
            def _():
                idxload(i + 3, b)

        return 0

    lax.fori_loop(0, (NCH + 2) // 3, tri, 0)
    plsc.subcore_barrier()

    @pl.when(s < NS - 1)
    def _():
        pltpu.sync_copy(acc.at[pl.ds(row0, NPT)],
                        out_hbm.at[c, pl.ds(row0, NPT)])

    @pl.when(s == NS - 1)
    def _():
        pltpu.sync_copy(acc.at[pl.ds(row0, NPT_LAST)],
                        out_hbm.at[c, pl.ds(row0, NPT_LAST)])


_spmm_kernel = functools.partial(
    pl.kernel,
    out_type=jax.ShapeDtypeStruct((NC, N, D), jnp.float32),
    mesh=_MESH,
    compiler_params=_SC_PARAMS,
    scratch_types=[
        pltpu.VMEM((CH,), jnp.int32),
        pltpu.VMEM((CH,), jnp.int32),
        pltpu.VMEM((CH,), jnp.int32),
        pltpu.VMEM((CH,), jnp.int32),
        pltpu.VMEM((CH,), jnp.int32),
        pltpu.VMEM((CH,), jnp.int32),
        pltpu.VMEM((CH, D), jnp.float32),
        pltpu.VMEM((CH, D), jnp.float32),
        pltpu.VMEM((CH, D), jnp.float32),
        pltpu.SemaphoreType.DMA,
        pltpu.SemaphoreType.DMA,
        pltpu.SemaphoreType.DMA,
        pltpu.SemaphoreType.DMA,
        pltpu.SemaphoreType.DMA,
        pltpu.SemaphoreType.DMA,
        pltpu.VMEM_SHARED((N, D), jnp.float32),
    ],
)(_spmm_body)


_RB = 1024
_GRID = (N + _RB - 1) // _RB
_RBE = 1000


def _dinv_of(parts_ref):
    i = pl.program_id(0)
    off = pl.multiple_of(i * _RB, 128)
    deg = 1.0 + jnp.sum(parts_ref[:, pl.ds(off, _RB)], axis=0)
    return lax.rsqrt(deg)


def _k2_body(parts_ref, x_ref, w_ref, hs_ref, dinv8_ref):
    dinv = _dinv_of(parts_ref)
    h = jnp.dot(x_ref[...], w_ref[...], preferred_element_type=jnp.float32)
    hs_ref[...] = h * dinv[:, None]
    dinv8_ref[...] = jnp.broadcast_to(dinv[:, None], (_RB, 8))


def _k4_body(p_ref, dinv8_ref, b_ref, hs2_ref):
    dinv = dinv8_ref[:, 0]
    acc = p_ref[0] + p_ref[1]
    h = jnp.maximum(acc * dinv[:, None] + b_ref[...], 0.0)
    hs2_ref[...] = h * dinv[:, None]


def _k6_body(q_ref, dinv8_ref, wmu_ref, bmu_ref, wls_ref, bls_ref,
             mu_ref, ls_ref):
    dinv = dinv8_ref[:, 0]
    g = (q_ref[0] + q_ref[1]) * dinv[:, None]
    mu_ref[...] = jnp.dot(g, wmu_ref[...],
                          preferred_element_type=jnp.float32) + bmu_ref[...]
    ls_ref[...] = jnp.dot(g, wls_ref[...],
                          preferred_element_type=jnp.float32) + bls_ref[...]


def _k2(parts, x, W1):
    return pl.pallas_call(
        _k2_body,
        grid=(_GRID,),
        in_specs=[
            pl.BlockSpec((NW, NP), lambda i: (0, 0)),
            pl.BlockSpec((_RB, D), lambda i: (i, 0)),
            pl.BlockSpec((D, D), lambda i: (0, 0)),
        ],
        out_specs=[
            pl.BlockSpec((_RB, D), lambda i: (i, 0)),
            pl.BlockSpec((_RB, 8), lambda i: (i, 0)),
        ],
        out_shape=[
            jax.ShapeDtypeStruct((N, D), jnp.float32),
            jax.ShapeDtypeStruct((N, 8), jnp.float32),
        ],
    )(parts, x, W1)


def _k4(p, dinv8, b1):
    return pl.pallas_call(
        _k4_body,
        grid=(N // _RBE,),
        in_specs=[
            pl.BlockSpec((NC, _RBE, D), lambda i: (0, i, 0)),
            pl.BlockSpec((_RBE, 8), lambda i: (i, 0)),
            pl.BlockSpec((1, D), lambda i: (0, 0)),
        ],
        out_specs=pl.BlockSpec((_RBE, D), lambda i: (i, 0)),
        out_shape=jax.ShapeDtypeStruct((N, D), jnp.float32),
    )(p, dinv8, b1)


def _k6(q, dinv8, W_mu, b_mu, W_ls, b_ls):
    return pl.pallas_call(
        _k6_body,
        grid=(N // _RBE,),
        in_specs=[
            pl.BlockSpec((NC, _RBE, D), lambda i: (0, i, 0)),
            pl.BlockSpec((_RBE, 8), lambda i: (i, 0)),
            pl.BlockSpec((D, DO), lambda i: (0, 0)),
            pl.BlockSpec((1, DO), lambda i: (0, 0)),
            pl.BlockSpec((D, DO), lambda i: (0, 0)),
            pl.BlockSpec((1, DO), lambda i: (0, 0)),
        ],
        out_specs=[
            pl.BlockSpec((_RBE, DO), lambda i: (i, 0)),
            pl.BlockSpec((_RBE, DO), lambda i: (i, 0)),
        ],
        out_shape=[
            jax.ShapeDtypeStruct((N, DO), jnp.float32),
            jax.ShapeDtypeStruct((N, DO), jnp.float32),
        ],
    )(q, dinv8, W_mu, b_mu, W_ls, b_ls)


def kernel(x, edge_index, W1, b1, W_mu, b_mu, W_ls, b_ls):
    ei = edge_index.astype(jnp.int32).reshape(2 * E)

    parts = _deg_kernel(ei)
    hs, dinv8 = _k2(parts, x, W1)
    p = _spmm_kernel(hs, ei)
    hs2 = _k4(p, dinv8, b1.reshape(1, D))
    q = _spmm_kernel(hs2, ei)
    mu, ls = _k6(q, dinv8, W_mu, b_mu.reshape(1, DO), W_ls, b_ls.reshape(1, DO))
    return mu, ls


# speedup vs baseline: 1.2712x; 1.2712x over previous
import functools

import jax
import jax.numpy as jnp
from jax import lax
from jax.experimental import pallas as pl
from jax.experimental.pallas import tpu as pltpu
from jax.experimental.pallas import tpu_sc as plsc

N = 10000
NP = 10240
E = 320000
D = 128
DO = 64

NC = 2
NS = 16
L = 16
NW = NC * NS
EPW = E // NW
NPT = 640
NPT_LAST = N - (NS - 1) * NPT
CH = 80
NCH = EPW // CH
PH_A = 75
TBL = PH_A * CH

_MESH = plsc.VectorSubcoreMesh(core_axis_name="c", subcore_axis_name="s")
_SC_PARAMS = pltpu.CompilerParams(needs_layout_passes=False)


def _deg_body(ei_hbm, parts_hbm, dst_v, deg_v):
    c = lax.axis_index("c")
    s = lax.axis_index("s")
    wid = s * NC + c

    zl = jnp.zeros((L,), jnp.float32)

    def zero(i, _):
        for k in range(4):
            deg_v[pl.ds((i * 4 + k) * L, L)] = zl
        return 0

    lax.fori_loop(0, NP // L // 4, zero, 0)

    pltpu.sync_copy(ei_hbm.at[pl.ds(E + wid * EPW, EPW)], dst_v)
    ones = jnp.ones((L,), jnp.float32)

    def scat(i, _):
        for k in range(5):
            idx = dst_v[pl.ds((i * 5 + k) * L, L)]
            plsc.addupdate_scatter(deg_v, [idx], ones)
        return 0

    lax.fori_loop(0, EPW // L // 5, scat, 0)
    pltpu.sync_copy(deg_v, parts_hbm.at[wid])


_deg_kernel = functools.partial(
    pl.kernel,
    out_type=jax.ShapeDtypeStruct((NW, NP), jnp.float32),
    mesh=_MESH,
    compiler_params=_SC_PARAMS,
    scratch_types=[
        pltpu.VMEM((EPW,), jnp.int32),
        pltpu.VMEM((NP,), jnp.float32),
    ],
)(_deg_body)


def _spmm_body(hs_hbm, ei_hbm, out_hbm,
               src_v, dst0_v, dst1_v, dst2_v,
               rows0_v, rows1_v, rows2_v,
               gsem0, gsem1, gsem2, isem0, isem1, isem2, acc):
    c = lax.axis_index("c")
    s = lax.axis_index("s")
    wid = s * NC + c
    row0 = pl.multiple_of(s * NPT, 8)


    @pl.when(jnp.logical_and(c == 0, s < NS - 1))
    def _():
        pltpu.sync_copy(hs_hbm.at[pl.ds(row0, NPT)], acc.at[pl.ds(row0, NPT)])

    @pl.when(jnp.logical_and(c == 0, s == NS - 1))
    def _():
        pltpu.sync_copy(hs_hbm.at[pl.ds(row0, NPT_LAST)],
                        acc.at[pl.ds(row0, NPT_LAST)])

    @pl.when(c != 0)
    def _():
        zl = jnp.zeros((L,), jnp.float32)

        def zrow(i, _):
            r = i // (D // L)
            k = i % (D // L)
            rows0_v[r, pl.ds(k * L, L)] = zl
            return 0

        lax.fori_loop(0, CH * D // L, zrow, 0)

    @pl.when(jnp.logical_and(c != 0, s < NS - 1))
    def _():
        for t in range(NPT // CH):
            r = pl.multiple_of(row0 + t * CH, 8)
            pltpu.sync_copy(rows0_v, acc.at[pl.ds(r, CH)])

    @pl.when(jnp.logical_and(c != 0, s == NS - 1))
    def _():
        for t in range(NPT_LAST // CH):
            r = pl.multiple_of(row0 + t * CH, 8)
            pltpu.sync_copy(rows0_v, acc.at[pl.ds(r, CH)])

    plsc.subcore_barrier()

    bufs = (rows0_v, rows1_v, rows2_v)
    gsems = (gsem0, gsem1, gsem2)
    dbufs = (dst0_v, dst1_v, dst2_v)
    isems = (isem0, isem1, isem2)

    def dstload(i, b):
        off = pl.multiple_of(E + wid * EPW + i * CH, 8)
        return pltpu.make_async_copy(ei_hbm.at[pl.ds(off, CH)],
                                     dbufs[b], isems[b])

    def run_phase(c0, nch_p):
        soff = pl.multiple_of(wid * EPW + c0 * CH, 8)
        pltpu.sync_copy(ei_hbm.at[pl.ds(soff, nch_p * CH)],
                        src_v.at[pl.ds(0, nch_p * CH)])

        def gather(l, b):
            return pltpu.make_async_copy(
                hs_hbm.at[src_v.at[pl.ds(l * CH, CH)]], bufs[b], gsems[b])

        dstload(c0, 0).start()
        dstload(c0 + 1, 1).start()
        gather(0, 0).start()
        gather(1, 1).start()
        dstload(c0 + 2, 2).start()

        def tri(j, _):
            for k in range(3):
                l = j * 3 + k
                b, b2 = k, (k + 2) % 3

                @pl.when(l + 2 < nch_p)
                def _():
                    gather(l + 2, b2).start()

                @pl.when(l < nch_p)
                def _():
                    gather(l, b).wait()
                    dstload(c0 + l, b).wait()
                    pltpu.sync_copy(bufs[b], acc.at[dbufs[b]], add=True)

                @pl.when(l + 3 < nch_p)
                def _():
                    dstload(c0 + l + 3, b).start()

            return 0

        lax.fori_loop(0, (nch_p + 2) // 3, tri, 0)

    run_phase(0, PH_A)
    run_phase(PH_A, NCH - PH_A)
    plsc.subcore_barrier()

    @pl.when(s < NS - 1)
    def _():
        pltpu.sync_copy(acc.at[pl.ds(row0, NPT)],
                        out_hbm.at[c, pl.ds(row0, NPT)])

    @pl.when(s == NS - 1)
    def _():
        pltpu.sync_copy(acc.at[pl.ds(row0, NPT_LAST)],
                        out_hbm.at[c, pl.ds(row0, NPT_LAST)])


_spmm_kernel = functools.partial(
    pl.kernel,
    out_type=jax.ShapeDtypeStruct((NC, N, D), jnp.float32),
    mesh=_MESH,
    compiler_params=_SC_PARAMS,
    scratch_types=[
        pltpu.VMEM((TBL,), jnp.int32),
        pltpu.VMEM((CH,), jnp.int32),
        pltpu.VMEM((CH,), jnp.int32),
        pltpu.VMEM((CH,), jnp.int32),
        pltpu.VMEM((CH, D), jnp.float32),
        pltpu.VMEM((CH, D), jnp.float32),
        pltpu.VMEM((CH, D), jnp.float32),
        pltpu.SemaphoreType.DMA,
        pltpu.SemaphoreType.DMA,
        pltpu.SemaphoreType.DMA,
        pltpu.SemaphoreType.DMA,
        pltpu.SemaphoreType.DMA,
        pltpu.SemaphoreType.DMA,
        pltpu.VMEM_SHARED((N, D), jnp.float32),
    ],
)(_spmm_body)


_RB = 1024
_GRID = (N + _RB - 1) // _RB
_RBE = 1000


def _dinv_of(parts_ref):
    i = pl.program_id(0)
    off = pl.multiple_of(i * _RB, 128)
    deg = 1.0 + jnp.sum(parts_ref[:, pl.ds(off, _RB)], axis=0)
    return lax.rsqrt(deg)


def _k2_body(parts_ref, x_ref, w_ref, hs_ref, dinv8_ref):
    dinv = _dinv_of(parts_ref)
    h = jnp.dot(x_ref[...], w_ref[...], preferred_element_type=jnp.float32)
    hs_ref[...] = h * dinv[:, None]
    dinv8_ref[...] = jnp.broadcast_to(dinv[:, None], (_RB, 8))


def _k4_body(p_ref, dinv8_ref, b_ref, hs2_ref):
    dinv = dinv8_ref[:, 0]
    acc = p_ref[0] + p_ref[1]
    h = jnp.maximum(acc * dinv[:, None] + b_ref[...], 0.0)
    hs2_ref[...] = h * dinv[:, None]


def _k6_body(q_ref, dinv8_ref, wmu_ref, bmu_ref, wls_ref, bls_ref,
             mu_ref, ls_ref):
    dinv = dinv8_ref[:, 0]
    g = (q_ref[0] + q_ref[1]) * dinv[:, None]
    mu_ref[...] = jnp.dot(g, wmu_ref[...],
                          preferred_element_type=jnp.float32) + bmu_ref[...]
    ls_ref[...] = jnp.dot(g, wls_ref[...],
                          preferred_element_type=jnp.float32) + bls_ref[...]


def _k2(parts, x, W1):
    return pl.pallas_call(
        _k2_body,
        grid=(_GRID,),
        in_specs=[
            pl.BlockSpec((NW, NP), lambda i: (0, 0)),
            pl.BlockSpec((_RB, D), lambda i: (i, 0)),
            pl.BlockSpec((D, D), lambda i: (0, 0)),
        ],
        out_specs=[
            pl.BlockSpec((_RB, D), lambda i: (i, 0)),
            pl.BlockSpec((_RB, 8), lambda i: (i, 0)),
        ],
        out_shape=[
            jax.ShapeDtypeStruct((N, D), jnp.float32),
            jax.ShapeDtypeStruct((N, 8), jnp.float32),
        ],
    )(parts, x, W1)


def _k4(p, dinv8, b1):
    return pl.pallas_call(
        _k4_body,
        grid=(N // _RBE,),
        in_specs=[
            pl.BlockSpec((NC, _RBE, D), lambda i: (0, i, 0)),
            pl.BlockSpec((_RBE, 8), lambda i: (i, 0)),
            pl.BlockSpec((1, D), lambda i: (0, 0)),
        ],
        out_specs=pl.BlockSpec((_RBE, D), lambda i: (i, 0)),
        out_shape=jax.ShapeDtypeStruct((N, D), jnp.float32),
    )(p, dinv8, b1)


def _k6(q, dinv8, W_mu, b_mu, W_ls, b_ls):
    return pl.pallas_call(
        _k6_body,
        grid=(N // _RBE,),
        in_specs=[
            pl.BlockSpec((NC, _RBE, D), lambda i: (0, i, 0)),
            pl.BlockSpec((_RBE, 8), lambda i: (i, 0)),
            pl.BlockSpec((D, DO), lambda i: (0, 0)),
            pl.BlockSpec((1, DO), lambda i: (0, 0)),
            pl.BlockSpec((D, DO), lambda i: (0, 0)),
            pl.BlockSpec((1, DO), lambda i: (0, 0)),
        ],
        out_specs=[
            pl.BlockSpec((_RBE, DO), lambda i: (i, 0)),
            pl.BlockSpec((_RBE, DO), lambda i: (i, 0)),
        ],
        out_shape=[
            jax.ShapeDtypeStruct((N, DO), jnp.float32),
            jax.ShapeDtypeStruct((N, DO), jnp.float32),
        ],
    )(q, dinv8, W_mu, b_mu, W_ls, b_ls)


def kernel(x, edge_index, W1, b1, W_mu, b_mu, W_ls, b_ls):
    ei = edge_index.astype(jnp.int32).reshape(2 * E)

    parts = _deg_kernel(ei)
    hs, dinv8 = _k2(parts, x, W1)
    p = _spmm_kernel(hs, ei)
    hs2 = _k4(p, dinv8, b1.reshape(1, D))
    q = _spmm_kernel(hs2, ei)
    mu, ls = _k6(q, dinv8, W_mu, b_mu.reshape(1, DO), W_ls, b_ls.reshape(1, DO))
    return mu, ls


# speedup vs baseline: 1.2728x; 1.0013x over previous
import functools

import jax
import jax.numpy as jnp
from jax import lax
from jax.experimental import pallas as pl
from jax.experimental.pallas import tpu as pltpu
from jax.experimental.pallas import tpu_sc as plsc

N = 10000
NP = 10240
E = 320000
D = 128
DO = 64

NC = 2
NS = 16
L = 16
NW = NC * NS
EPW = E // NW
NPT = 640
NPT_LAST = N - (NS - 1) * NPT
CH = 80
NCH = EPW // CH
PH_A = 75
TBL = PH_A * CH

_MESH = plsc.VectorSubcoreMesh(core_axis_name="c", subcore_axis_name="s")
_SC_PARAMS = pltpu.CompilerParams(needs_layout_passes=False)


def _deg_body(ei_hbm, parts_hbm, dst_v, deg_v):
    c = lax.axis_index("c")
    s = lax.axis_index("s")
    wid = s * NC + c

    zl = jnp.zeros((L,), jnp.float32)

    def zero(i, _):
        for k in range(4):
            deg_v[pl.ds((i * 4 + k) * L, L)] = zl
        return 0

    lax.fori_loop(0, NP // L // 4, zero, 0)

    pltpu.sync_copy(ei_hbm.at[pl.ds(E + wid * EPW, EPW)], dst_v)
    ones = jnp.ones((L,), jnp.float32)

    def scat(i, _):
        for k in range(5):
            idx = dst_v[pl.ds((i * 5 + k) * L, L)]
            plsc.addupdate_scatter(deg_v, [idx], ones)
        return 0

    lax.fori_loop(0, EPW // L // 5, scat, 0)
    pltpu.sync_copy(deg_v, parts_hbm.at[wid])


_deg_kernel = functools.partial(
    pl.kernel,
    out_type=jax.ShapeDtypeStruct((NW, NP), jnp.float32),
    mesh=_MESH,
    compiler_params=_SC_PARAMS,
    scratch_types=[
        pltpu.VMEM((EPW,), jnp.int32),
        pltpu.VMEM((NP,), jnp.float32),
    ],
)(_deg_body)


def _spmm_body(hs_hbm, ei_hbm, out_hbm,
               src_v, dst0_v, dst1_v, dst2_v,
               rows0_v, rows1_v, rows2_v,
               gsem0, gsem1, gsem2, isem0, isem1, isem2, acc):
    c = lax.axis_index("c")
    s = lax.axis_index("s")
    wid = s * NC + c
    row0 = pl.multiple_of(s * NPT, 8)


    @pl.when(jnp.logical_and(c == 0, s < NS - 1))
    def _():
        pltpu.sync_copy(hs_hbm.at[pl.ds(row0, NPT)], acc.at[pl.ds(row0, NPT)])

    @pl.when(jnp.logical_and(c == 0, s == NS - 1))
    def _():
        pltpu.sync_copy(hs_hbm.at[pl.ds(row0, NPT_LAST)],
                        acc.at[pl.ds(row0, NPT_LAST)])

    @pl.when(c != 0)
    def _():
        zl = jnp.zeros((L,), jnp.float32)

        def zrow(i, _):
            r = i // (D // L)
            k = i % (D // L)
            rows0_v[r, pl.ds(k * L, L)] = zl
            return 0

        lax.fori_loop(0, CH * D // L, zrow, 0)

    @pl.when(jnp.logical_and(c != 0, s < NS - 1))
    def _():
        for t in range(NPT // CH):
            r = pl.multiple_of(row0 + t * CH, 8)
            pltpu.sync_copy(rows0_v, acc.at[pl.ds(r, CH)])

    @pl.when(jnp.logical_and(c != 0, s == NS - 1))
    def _():
        for t in range(NPT_LAST // CH):
            r = pl.multiple_of(row0 + t * CH, 8)
            pltpu.sync_copy(rows0_v, acc.at[pl.ds(r, CH)])

    plsc.subcore_barrier()

    bufs = (rows0_v, rows1_v, rows2_v)
    gsems = (gsem0, gsem1, gsem2)
    dbufs = (dst0_v, dst1_v, dst2_v)
    isems = (isem0, isem1, isem2)

    def dstload(i, b):
        off = pl.multiple_of(E + wid * EPW + i * CH, 8)
        return pltpu.make_async_copy(ei_hbm.at[pl.ds(off, CH)],
                                     dbufs[b], isems[b])

    def run_phase(c0, nch_p):
        soff = pl.multiple_of(wid * EPW + c0 * CH, 8)
        pltpu.sync_copy(ei_hbm.at[pl.ds(soff, nch_p * CH)],
                        src_v.at[pl.ds(0, nch_p * CH)])

        H = CH // 2

        def gather_h(l, b, h):
            return pltpu.make_async_copy(
                hs_hbm.at[src_v.at[pl.ds(l * CH + h * H, H)]],
                bufs[b].at[pl.ds(h * H, H)], gsems[b])

        class gather:
            def __init__(self, l, b):
                self.l, self.b = l, b

            def start(self):
                gather_h(self.l, self.b, 0).start()
                gather_h(self.l, self.b, 1).start()

            def wait(self):
                gather_h(self.l, self.b, 0).wait()
                gather_h(self.l, self.b, 1).wait()

        dstload(c0, 0).start()
        dstload(c0 + 1, 1).start()
        gather(0, 0).start()
        gather(1, 1).start()
        dstload(c0 + 2, 2).start()

        def tri(j, _):
            for k in range(3):
                l = j * 3 + k
                b, b2 = k, (k + 2) % 3

                @pl.when(l + 2 < nch_p)
                def _():
                    gather(l + 2, b2).start()

                @pl.when(l < nch_p)
                def _():
                    gather(l, b).wait()
                    dstload(c0 + l, b).wait()
                    pltpu.sync_copy(bufs[b], acc.at[dbufs[b]], add=True)

                @pl.when(l + 3 < nch_p)
                def _():
                    dstload(c0 + l + 3, b).start()

            return 0

        lax.fori_loop(0, (nch_p + 2) // 3, tri, 0)

    run_phase(0, PH_A)
    run_phase(PH_A, NCH - PH_A)
    plsc.subcore_barrier()

    @pl.when(s < NS - 1)
    def _():
        pltpu.sync_copy(acc.at[pl.ds(row0, NPT)],
                        out_hbm.at[c, pl.ds(row0, NPT)])

    @pl.when(s == NS - 1)
    def _():
        pltpu.sync_copy(acc.at[pl.ds(row0, NPT_LAST)],
                        out_hbm.at[c, pl.ds(row0, NPT_LAST)])


_spmm_kernel = functools.partial(
    pl.kernel,
    out_type=jax.ShapeDtypeStruct((NC, N, D), jnp.float32),
    mesh=_MESH,
    compiler_params=_SC_PARAMS,
    scratch_types=[
        pltpu.VMEM((TBL,), jnp.int32),
        pltpu.VMEM((CH,), jnp.int32),
        pltpu.VMEM((CH,), jnp.int32),
        pltpu.VMEM((CH,), jnp.int32),
        pltpu.VMEM((CH, D), jnp.float32),
        pltpu.VMEM((CH, D), jnp.float32),
        pltpu.VMEM((CH, D), jnp.float32),
        pltpu.SemaphoreType.DMA,
        pltpu.SemaphoreType.DMA,
        pltpu.SemaphoreType.DMA,
        pltpu.SemaphoreType.DMA,
        pltpu.SemaphoreType.DMA,
        pltpu.SemaphoreType.DMA,
        pltpu.VMEM_SHARED((N, D), jnp.float32),
    ],
)(_spmm_body)


_RB = 1024
_GRID = (N + _RB - 1) // _RB
_RBE = 1000


def _dinv_of(parts_ref):
    i = pl.program_id(0)
    off = pl.multiple_of(i * _RB, 128)
    deg = 1.0 + jnp.sum(parts_ref[:, pl.ds(off, _RB)], axis=0)
    return lax.rsqrt(deg)


def _k2_body(parts_ref, x_ref, w_ref, hs_ref, dinv8_ref):
    dinv = _dinv_of(parts_ref)
    h = jnp.dot(x_ref[...], w_ref[...], preferred_element_type=jnp.float32)
    hs_ref[...] = h * dinv[:, None]
    dinv8_ref[...] = jnp.broadcast_to(dinv[:, None], (_RB, 8))


def _k4_body(p_ref, dinv8_ref, b_ref, hs2_ref):
    dinv = dinv8_ref[:, 0]
    acc = p_ref[0] + p_ref[1]
    h = jnp.maximum(acc * dinv[:, None] + b_ref[...], 0.0)
    hs2_ref[...] = h * dinv[:, None]


def _k6_body(q_ref, dinv8_ref, wmu_ref, bmu_ref, wls_ref, bls_ref,
             mu_ref, ls_ref):
    dinv = dinv8_ref[:, 0]
    g = (q_ref[0] + q_ref[1]) * dinv[:, None]
    mu_ref[...] = jnp.dot(g, wmu_ref[...],
                          preferred_element_type=jnp.float32) + bmu_ref[...]
    ls_ref[...] = jnp.dot(g, wls_ref[...],
                          preferred_element_type=jnp.float32) + bls_ref[...]


def _k2(parts, x, W1):
    return pl.pallas_call(
        _k2_body,
        grid=(_GRID,),
        in_specs=[
            pl.BlockSpec((NW, NP), lambda i: (0, 0)),
            pl.BlockSpec((_RB, D), lambda i: (i, 0)),
            pl.BlockSpec((D, D), lambda i: (0, 0)),
        ],
        out_specs=[
            pl.BlockSpec((_RB, D), lambda i: (i, 0)),
            pl.BlockSpec((_RB, 8), lambda i: (i, 0)),
        ],
        out_shape=[
            jax.ShapeDtypeStruct((N, D), jnp.float32),
            jax.ShapeDtypeStruct((N, 8), jnp.float32),
        ],
    )(parts, x, W1)


def _k4(p, dinv8, b1):
    return pl.pallas_call(
        _k4_body,
        grid=(N // _RBE,),
        in_specs=[
            pl.BlockSpec((NC, _RBE, D), lambda i: (0, i, 0)),
            pl.BlockSpec((_RBE, 8), lambda i: (i, 0)),
            pl.BlockSpec((1, D), lambda i: (0, 0)),
        ],
        out_specs=pl.BlockSpec((_RBE, D), lambda i: (i, 0)),
        out_shape=jax.ShapeDtypeStruct((N, D), jnp.float32),
    )(p, dinv8, b1)


def _k6(q, dinv8, W_mu, b_mu, W_ls, b_ls):
    return pl.pallas_call(
        _k6_body,
        grid=(N // _RBE,),
        in_specs=[
            pl.BlockSpec((NC, _RBE, D), lambda i: (0, i, 0)),
            pl.BlockSpec((_RBE, 8), lambda i: (i, 0)),
            pl.BlockSpec((D, DO), lambda i: (0, 0)),
            pl.BlockSpec((1, DO), lambda i: (0, 0)),
            pl.BlockSpec((D, DO), lambda i: (0, 0)),
            pl.BlockSpec((1, DO), lambda i: (0, 0)),
        ],
        out_specs=[
            pl.BlockSpec((_RBE, DO), lambda i: (i, 0)),
            pl.BlockSpec((_RBE, DO), lambda i: (i, 0)),
        ],
        out_shape=[
            jax.ShapeDtypeStruct((N, DO), jnp.float32),
            jax.ShapeDtypeStruct((N, DO), jnp.float32),
        ],
    )(q, dinv8, W_mu, b_mu, W_ls, b_ls)


def kernel(x, edge_index, W1, b1, W_mu, b_mu, W_ls, b_ls):
    ei = edge_index.astype(jnp.int32).reshape(2 * E)

    parts = _deg_kernel(ei)
    hs, dinv8 = _k2(parts, x, W1)
    p = _spmm_kernel(hs, ei)
    hs2 = _k4(p, dinv8, b1.reshape(1, D))
    q = _spmm_kernel(hs2, ei)
    mu, ls = _k6(q, dinv8, W_mu, b_mu.reshape(1, DO), W_ls, b_ls.reshape(1, DO))
    return mu, ls
